# Initial kernel scaffold; baseline (speedup 1.0000x reference)
import jax
import jax.numpy as jnp
from jax.experimental import pallas as pl


def kernel(hidden_states, gate_w, gate_proj_w, up_proj_w, down_proj_w):
    raise NotImplementedError("write your pallas kernel here")


# speedup vs baseline: 1.2200x; 1.2200x over previous
import functools

import jax
import jax.numpy as jnp
from jax.experimental import pallas as pl

HIDDEN = 768
INTER = 2048
NUM_EXPERTS = 64

ROUTER_BLOCK = 2048
FFN_IC = 512


def _router_body(x_ref, gw_ref, logits_ref, m_ref, s_ref, aux_ref):
    i = pl.program_id(0)

    @pl.when(i == 0)
    def _init():
        m_ref[...] = jnp.full_like(m_ref, -1e30)
        s_ref[...] = jnp.zeros_like(s_ref)
        aux_ref[...] = jnp.zeros_like(aux_ref)

    xb = x_ref[...]
    gw = gw_ref[...]
    lg = jax.lax.dot_general(xb, gw, (((1,), (1,)), ((), ())),
                             preferred_element_type=jnp.float32)
    logits_ref[...] = lg

    blkmax = jnp.max(lg, axis=0, keepdims=True)
    m_old = m_ref[...]
    m_new = jnp.maximum(m_old, blkmax)
    s_ref[...] = (s_ref[...] * jnp.exp(m_old - m_new)
                  + jnp.sum(jnp.exp(lg - m_new), axis=0, keepdims=True))
    m_ref[...] = m_new

    tmax = jnp.max(lg, axis=1, keepdims=True)
    lse = jnp.log(jnp.sum(jnp.exp(lg - tmax), axis=1, keepdims=True)) + tmax
    aux_ref[...] += jnp.sum(lse * lse).reshape(1, 1)


def _router(x, gate_w):
    n, h = x.shape
    e = gate_w.shape[0]
    nblocks = n // ROUTER_BLOCK
    return pl.pallas_call(
        _router_body,
        grid=(nblocks,),
        in_specs=[
            pl.BlockSpec((ROUTER_BLOCK, h), lambda i: (i, 0)),
            pl.BlockSpec((e, h), lambda i: (0, 0)),
        ],
        out_specs=[
            pl.BlockSpec((ROUTER_BLOCK, e), lambda i: (i, 0)),
            pl.BlockSpec((1, e), lambda i: (0, 0)),
            pl.BlockSpec((1, e), lambda i: (0, 0)),
            pl.BlockSpec((1, 1), lambda i: (0, 0)),
        ],
        out_shape=[
            jax.ShapeDtypeStruct((n, e), jnp.float32),
            jax.ShapeDtypeStruct((1, e), jnp.float32),
            jax.ShapeDtypeStruct((1, e), jnp.float32),
            jax.ShapeDtypeStruct((1, 1), jnp.float32),
        ],
    )(x, gate_w)


def _ffn_body(nk, xe_ref, wg_ref, wu_ref, wd_ref, p_ref, out_ref):
    k = pl.program_id(1)
    xe = xe_ref[0]
    wg = wg_ref[0]
    wu = wu_ref[0]
    g = jax.lax.dot_general(xe, wg, (((1,), (1,)), ((), ())),
                            preferred_element_type=jnp.float32)
    u = jax.lax.dot_general(xe, wu, (((1,), (1,)), ((), ())),
                            preferred_element_type=jnp.float32)
    hact = (g * jax.nn.sigmoid(g)) * u
    wd = wd_ref[0]
    o = jax.lax.dot_general(hact, wd, (((1,), (1,)), ((), ())),
                            preferred_element_type=jnp.float32)

    @pl.when(k == 0)
    def _():
        out_ref[0] = o

    @pl.when(k > 0)
    def _():
        out_ref[0] += o

    @pl.when(k == nk - 1)
    def _():
        out_ref[0] *= p_ref[0]


def _ffn(expert_in, gate_proj_w, up_proj_w, down_proj_w, top_probs):
    e, cap, h = expert_in.shape
    inter = gate_proj_w.shape[1]
    nk = inter // FFN_IC
    return pl.pallas_call(
        functools.partial(_ffn_body, nk),
        grid=(e, nk),
        in_specs=[
            pl.BlockSpec((1, cap, h), lambda ei, k: (ei, 0, 0)),
            pl.BlockSpec((1, FFN_IC, h), lambda ei, k: (ei, k, 0)),
            pl.BlockSpec((1, FFN_IC, h), lambda ei, k: (ei, k, 0)),
            pl.BlockSpec((1, h, FFN_IC), lambda ei, k: (ei, 0, k)),
            pl.BlockSpec((1, cap, 1), lambda ei, k: (ei, 0, 0)),
        ],
        out_specs=pl.BlockSpec((1, cap, h), lambda ei, k: (ei, 0, 0)),
        out_shape=jax.ShapeDtypeStruct((e, cap, h), jnp.float32),
    )(expert_in, gate_proj_w, up_proj_w, down_proj_w, top_probs[..., None])


def kernel(hidden_states, gate_w, gate_proj_w, up_proj_w, down_proj_w):
    b, seq, h = hidden_states.shape
    x = hidden_states.reshape(-1, h)
    n = x.shape[0]
    e = gate_w.shape[0]
    cap = max(n // e, 1)
    cap = min(cap, n)

    logits, m, s, aux_sum = _router(x, gate_w)

    top_logits, top_idx = jax.lax.top_k(logits.T, cap)
    top_probs = jnp.exp(top_logits - m.reshape(e, 1)) / s.reshape(e, 1)

    expert_in = x[top_idx]
    weighted = _ffn(expert_in, gate_proj_w, up_proj_w, down_proj_w, top_probs)

    flat_idx = top_idx.reshape(-1)
    final = jnp.zeros_like(x).at[flat_idx].add(weighted.reshape(-1, h))
    token_counts = jnp.zeros((n,), x.dtype).at[flat_idx].add(top_probs.reshape(-1))
    token_counts = jnp.clip(token_counts, 1e-9, None)
    final = (final / token_counts[:, None]).reshape(b, seq, h)

    aux_loss = (aux_sum.reshape(()) / n) * 0.001
    return final, aux_loss
